# baseline (speedup 1.0000x reference)
import functools

import jax
import jax.numpy as jnp
from jax import lax
from jax.experimental import pallas as pl
from jax.experimental.pallas import tpu as pltpu
from jax.experimental.pallas import tpu_sc as plsc

N = 100000
D = 128
E = 300000

NS = 16
LR = 8
NCH = D // LR
CH = 128
EPT = 18816
NCHUNKS = EPT // CH
NPAD = 100096
ROWS_PT = NPAD // NS
ZR = 2048


def _sc_body(xs_hbm, xd_hbm, e1s_hbm, e1d_hbm, e2s_hbm, e2d_hbm, cst_hbm,
             out1, out2, src_idx, dst_idx, rows, ones, zeros, acc, sem):
    c = lax.axis_index("c")
    s = lax.axis_index("s")

    pltpu.sync_copy(cst_hbm.at[pl.ds(0, ZR)], zeros)
    pltpu.sync_copy(cst_hbm.at[pl.ds(ZR, CH)], ones)

    @pl.when(c == 0)
    def _():
        pltpu.sync_copy(e1s_hbm.at[s], src_idx)
        pltpu.sync_copy(e1d_hbm.at[s], dst_idx)

    @pl.when(c == 1)
    def _():
        pltpu.sync_copy(e2s_hbm.at[s], src_idx)
        pltpu.sync_copy(e2d_hbm.at[s], dst_idx)

    def run(xtab, out):
        base = s * ROWS_PT
        for p in range(NCH + 1):
            for k in range(ROWS_PT // ZR):
                pltpu.sync_copy(zeros, acc.at[pl.ds(base + k * ZR, ZR)])
            rem = ROWS_PT % ZR
            if rem:
                pltpu.sync_copy(zeros.at[pl.ds(0, rem)],
                                acc.at[pl.ds(base + ROWS_PT - rem, rem)])
            plsc.subcore_barrier()

            if p < NCH:
                def chunk(j, carry):
                    pltpu.async_copy(xtab.at[p].at[src_idx.at[j]], rows,
                                     sem).wait()
                    pltpu.sync_copy(rows, acc.at[dst_idx.at[j]], add=True)
                    return carry

                lax.fori_loop(0, NCHUNKS, chunk, 0)
            else:
                def chunk(j, carry):
                    pltpu.sync_copy(ones, acc.at[dst_idx.at[j]], add=True)
                    return carry

                lax.fori_loop(0, NCHUNKS, chunk, 0)
            plsc.subcore_barrier()

            pltpu.sync_copy(acc.at[pl.ds(base, ROWS_PT)],
                            out.at[p].at[pl.ds(base, ROWS_PT)])
            plsc.subcore_barrier()

    @pl.when(c == 0)
    def _():
        run(xs_hbm, out1)

    @pl.when(c == 1)
    def _():
        run(xd_hbm, out2)


@jax.jit
def _sc_aggregate(xs_t, xd_t, e1s, e1d, e2s, e2d, cst):
    mesh = plsc.VectorSubcoreMesh(core_axis_name="c", subcore_axis_name="s")
    f = functools.partial(
        pl.kernel,
        out_type=(jax.ShapeDtypeStruct((NCH + 1, NPAD, LR), jnp.float32),
                  jax.ShapeDtypeStruct((NCH + 1, NPAD, LR), jnp.float32)),
        mesh=mesh,
        scratch_types=[
            pltpu.VMEM((NCHUNKS, CH), jnp.int32),
            pltpu.VMEM((NCHUNKS, CH), jnp.int32),
            pltpu.VMEM((CH, LR), jnp.float32),
            pltpu.VMEM((CH, LR), jnp.float32),
            pltpu.VMEM((ZR, LR), jnp.float32),
            pltpu.VMEM_SHARED((NPAD, LR), jnp.float32),
            pltpu.SemaphoreType.DMA,
        ],
        compiler_params=pltpu.CompilerParams(use_tc_tiling_on_sc=False),
    )(_sc_body)
    return f(xs_t, xd_t, e1s, e1d, e2s, e2d, cst)


def _dense_body(agg_ref, deg_ref, w_ref, b_ref, o_ref):
    x = agg_ref[...]
    cat = jnp.concatenate([x[k] for k in range(NCH)], axis=1)
    deg = jnp.maximum(deg_ref[...][0, :, 0:1], 1.0)
    h = cat / deg
    o_ref[...] = (
        jnp.dot(h, w_ref[...], preferred_element_type=jnp.float32)
        + b_ref[...]
    )


NB = 2176


@jax.jit
def _dense(agg, w, b):
    return pl.pallas_call(
        _dense_body,
        grid=(NPAD // NB,),
        in_specs=[
            pl.BlockSpec((NCH, NB, LR), lambda i: (0, i, 0)),
            pl.BlockSpec((1, NB, LR), lambda i: (NCH, i, 0)),
            pl.BlockSpec((D, D), lambda i: (0, 0)),
            pl.BlockSpec((1, D), lambda i: (0, 0)),
        ],
        out_specs=pl.BlockSpec((NB, D), lambda i: (i, 0)),
        out_shape=jax.ShapeDtypeStruct((NPAD, D), jnp.float32),
    )(agg, agg, w, b.reshape(1, D))


def _prep_edges(edge_index):
    pad = NS * EPT - E
    src = jnp.concatenate([edge_index[0], jnp.zeros((pad,), jnp.int32)])
    dst = jnp.concatenate([edge_index[1], jnp.full((pad,), N, jnp.int32)])
    return (src.reshape(NS, NCHUNKS, CH), dst.reshape(NS, NCHUNKS, CH))


def kernel(x_src, x_dst, edge_index_e1, edge_index_e2, W_e1, b_e1, W_e2, b_e2):
    xs_t = x_src.reshape(N, NCH, LR).transpose(1, 0, 2)
    xd_t = x_dst.reshape(N, NCH, LR).transpose(1, 0, 2)
    e1s, e1d = _prep_edges(edge_index_e1)
    e2s, e2d = _prep_edges(edge_index_e2)
    cst = jnp.concatenate([jnp.zeros((ZR, LR), jnp.float32),
                           jnp.ones((CH, LR), jnp.float32)])
    agg1, agg2 = _sc_aggregate(xs_t, xd_t, e1s, e1d, e2s, e2d, cst)
    out_dst = _dense(agg1, W_e1, b_e1)[:N]
    out_src = _dense(agg2, W_e2, b_e2)[:N]
    return (out_src, out_dst)


# speedup vs baseline: 2.6433x; 2.6433x over previous
import functools

import jax
import jax.numpy as jnp
from jax import lax
from jax.experimental import pallas as pl
from jax.experimental.pallas import tpu as pltpu
from jax.experimental.pallas import tpu_sc as plsc

N = 100000
D = 128
E = 300000

NS = 16
LR = 8
NCH = D // LR
CH = 128
NCHUNKS = 148
EPT = NCHUNKS * CH
NPAD = 100096
ROWS_PT = NPAD // NS
ZR = 2048


def _sc_body(xs_hbm, xd_hbm, e1s_hbm, e1d_hbm, e2s_hbm, e2d_hbm, cst_hbm,
             agg1, deg1, agg2, deg2,
             src_idx, dst_idx, rows_a, rows_b, ones, zeros,
             acc, sem_a, sem_b):
    c = lax.axis_index("c")
    s = lax.axis_index("s")

    pltpu.sync_copy(cst_hbm.at[pl.ds(0, ZR)], zeros)
    pltpu.sync_copy(cst_hbm.at[pl.ds(ZR, CH)], ones)

    @pl.when(c == 0)
    def _():
        pltpu.sync_copy(e1s_hbm.at[s], src_idx)
        pltpu.sync_copy(e1d_hbm.at[s], dst_idx)

    @pl.when(c == 1)
    def _():
        pltpu.sync_copy(e2s_hbm.at[s], src_idx)
        pltpu.sync_copy(e2d_hbm.at[s], dst_idx)

    def scale16(j, carry):
        for k in range(CH // 16):
            v = src_idx[j, pl.ds(k * 16, 16)]
            src_idx[j, pl.ds(k * 16, 16)] = v * 16
        return carry

    lax.fori_loop(0, NCHUNKS, scale16, 0)

    def bump(j, carry):
        for k in range(CH // 16):
            v = src_idx[j, pl.ds(k * 16, 16)]
            src_idx[j, pl.ds(k * 16, 16)] = v + 1
        return carry

    def zero_slice(base):
        for k in range(ROWS_PT // ZR):
            pltpu.sync_copy(zeros, acc.at[pl.ds(base + k * ZR, ZR)])
        rem = ROWS_PT % ZR
        if rem:
            pltpu.sync_copy(zeros.at[pl.ds(0, rem)],
                            acc.at[pl.ds(base + ROWS_PT - rem, rem)])

    def run(xflat, agg, deg):
        base = s * ROWS_PT
        zero_slice(base)
        plsc.subcore_barrier()
        for p in range(NCH):
            pltpu.async_copy(xflat.at[src_idx.at[0]], rows_a, sem_a)

            def chunk(i, carry):
                @pl.when(i % 2 == 0)
                def _():
                    @pl.when(i + 1 < NCHUNKS)
                    def _():
                        pltpu.async_copy(xflat.at[src_idx.at[i + 1]],
                                         rows_b, sem_b)
                    pltpu.make_async_copy(xflat.at[src_idx.at[i]],
                                          rows_a, sem_a).wait()
                    pltpu.sync_copy(rows_a, acc.at[dst_idx.at[i]], add=True)

                @pl.when(i % 2 == 1)
                def _():
                    @pl.when(i + 1 < NCHUNKS)
                    def _():
                        pltpu.async_copy(xflat.at[src_idx.at[i + 1]],
                                         rows_a, sem_a)
                    pltpu.make_async_copy(xflat.at[src_idx.at[i]],
                                          rows_b, sem_b).wait()
                    pltpu.sync_copy(rows_b, acc.at[dst_idx.at[i]], add=True)

                return carry

            lax.fori_loop(0, NCHUNKS, chunk, 0)
            if p < NCH - 1:
                lax.fori_loop(0, NCHUNKS, bump, 0)
            plsc.subcore_barrier()
            pltpu.sync_copy(acc.at[pl.ds(base, ROWS_PT)],
                            agg.at[pl.ds(base, ROWS_PT), pl.ds(p * LR, LR)])
            zero_slice(base)
            plsc.subcore_barrier()

        def dchunk(i, carry):
            pltpu.sync_copy(ones, acc.at[dst_idx.at[i]], add=True)
            return carry

        lax.fori_loop(0, NCHUNKS, dchunk, 0)
        plsc.subcore_barrier()
        pltpu.sync_copy(acc.at[pl.ds(base, ROWS_PT)],
                        deg.at[pl.ds(base, ROWS_PT)])

    @pl.when(c == 0)
    def _():
        run(xs_hbm, agg1, deg1)

    @pl.when(c == 1)
    def _():
        run(xd_hbm, agg2, deg2)


@jax.jit
def _sc_aggregate(xs_flat, xd_flat, e1s, e1d, e2s, e2d, cst):
    mesh = plsc.VectorSubcoreMesh(core_axis_name="c", subcore_axis_name="s")
    f = functools.partial(
        pl.kernel,
        out_type=(jax.ShapeDtypeStruct((NPAD, D), jnp.float32),
                  jax.ShapeDtypeStruct((NPAD, LR), jnp.float32),
                  jax.ShapeDtypeStruct((NPAD, D), jnp.float32),
                  jax.ShapeDtypeStruct((NPAD, LR), jnp.float32)),
        mesh=mesh,
        scratch_types=[
            pltpu.VMEM((NCHUNKS, CH), jnp.int32),
            pltpu.VMEM((NCHUNKS, CH), jnp.int32),
            pltpu.VMEM((CH, LR), jnp.float32),
            pltpu.VMEM((CH, LR), jnp.float32),
            pltpu.VMEM((CH, LR), jnp.float32),
            pltpu.VMEM((ZR, LR), jnp.float32),
            pltpu.VMEM_SHARED((NPAD, LR), jnp.float32),
            pltpu.SemaphoreType.DMA,
            pltpu.SemaphoreType.DMA,
        ],
        compiler_params=pltpu.CompilerParams(use_tc_tiling_on_sc=False),
    )(_sc_body)
    return f(xs_flat, xd_flat, e1s, e1d, e2s, e2d, cst)


def _dense_body(agg_ref, deg_ref, w_ref, b_ref, o_ref):
    deg = jnp.maximum(deg_ref[...][:, 0:1], 1.0)
    h = agg_ref[...] / deg
    o_ref[...] = (
        jnp.dot(h, w_ref[...], preferred_element_type=jnp.float32)
        + b_ref[...]
    )


NB = 2176


@jax.jit
def _dense(agg, deg, w, b):
    return pl.pallas_call(
        _dense_body,
        grid=(NPAD // NB,),
        in_specs=[
            pl.BlockSpec((NB, D), lambda i: (i, 0)),
            pl.BlockSpec((NB, LR), lambda i: (i, 0)),
            pl.BlockSpec((D, D), lambda i: (0, 0)),
            pl.BlockSpec((1, D), lambda i: (0, 0)),
        ],
        out_specs=pl.BlockSpec((NB, D), lambda i: (i, 0)),
        out_shape=jax.ShapeDtypeStruct((N, D), jnp.float32),
    )(agg, deg, w, b.reshape(1, D))


def _prep_edges(edge_index):
    pad = NS * EPT - E
    src = jnp.concatenate([edge_index[0], jnp.zeros((pad,), jnp.int32)])
    dst = jnp.concatenate([edge_index[1], jnp.full((pad,), N, jnp.int32)])
    return (src.reshape(NS, NCHUNKS, CH), dst.reshape(NS, NCHUNKS, CH))


def kernel(x_src, x_dst, edge_index_e1, edge_index_e2, W_e1, b_e1, W_e2, b_e2):
    xs_flat = x_src.reshape(N * NCH, LR)
    xd_flat = x_dst.reshape(N * NCH, LR)
    e1s, e1d = _prep_edges(edge_index_e1)
    e2s, e2d = _prep_edges(edge_index_e2)
    cst = jnp.concatenate([jnp.zeros((ZR, LR), jnp.float32),
                           jnp.ones((CH, LR), jnp.float32)])
    agg1, deg1, agg2, deg2 = _sc_aggregate(xs_flat, xd_flat,
                                           e1s, e1d, e2s, e2d, cst)
    out_dst = _dense(agg1, deg1, W_e1, b_e1)
    out_src = _dense(agg2, deg2, W_e2, b_e2)
    return (out_src, out_dst)
